# baseline (speedup 1.0000x reference)
import functools

import jax
import jax.numpy as jnp
from jax import lax
from jax.experimental import pallas as pl
from jax.experimental.pallas import tpu as pltpu
from jax.experimental.pallas import tpu_sc as plsc

N_NODES = 10000
N_EDGES = 320000
N_PAD = 10240
NC, NS, LANES = 2, 16, 16
NW = NC * NS
BLK = 128
CH = 80
E_PAD = NW * CH * BLK
NBUF = 4
ROW_SLAB = N_PAD // NS

_MESH = plsc.VectorSubcoreMesh(core_axis_name="c", subcore_axis_name="s")


def _agg_call(y, src_p, dst_p, with_deg):
    dout = y.shape[1]
    scratch = [
        pltpu.VMEM((CH, BLK), jnp.int32),
        pltpu.VMEM((CH, BLK), jnp.int32),
        pltpu.VMEM((BLK, dout), jnp.float32),
        pltpu.VMEM_SHARED((N_PAD, dout), jnp.float32),
        pltpu.VMEM_SHARED((N_PAD, dout), jnp.float32),
        pltpu.SemaphoreType.DMA,
    ]
    outs = [jax.ShapeDtypeStruct((NC, N_PAD, dout), jnp.float32)]
    if with_deg:
        scratch += [
            pltpu.VMEM((BLK, LANES), jnp.float32),
            pltpu.VMEM_SHARED((N_PAD, LANES), jnp.float32),
        ]
        outs += [jax.ShapeDtypeStruct((NC, N_PAD, LANES), jnp.float32)]

    def body(*refs):
        if with_deg:
            (y_hbm, src_hbm, dst_hbm, zacc_hbm, zdeg_hbm, ones_hbm,
             acc_out, deg_out,
             src_v, dst_v, rows_v, acc_sh, y_sh, sem, ones_v, deg_sh) = refs
        else:
            (y_hbm, src_hbm, dst_hbm, zacc_hbm,
             acc_out,
             src_v, dst_v, rows_v, acc_sh, y_sh, sem) = refs
        c = lax.axis_index("c")
        s = lax.axis_index("s")
        wid = s * NC + c
        rbase = s * ROW_SLAB
        pltpu.sync_copy(src_hbm.at[wid], src_v)
        pltpu.sync_copy(dst_hbm.at[wid], dst_v)
        pltpu.sync_copy(y_hbm.at[pl.ds(rbase, ROW_SLAB)],
                        y_sh.at[pl.ds(rbase, ROW_SLAB)])
        for t in range(ROW_SLAB // BLK):
            pltpu.sync_copy(zacc_hbm, acc_sh.at[pl.ds(rbase + t * BLK, BLK)])
        if with_deg:
            for t in range(ROW_SLAB // BLK):
                pltpu.sync_copy(zdeg_hbm,
                                deg_sh.at[pl.ds(rbase + t * BLK, BLK)])
            pltpu.sync_copy(ones_hbm, ones_v)
        plsc.subcore_barrier()

        def step(j, carry):
            pltpu.async_copy(y_sh.at[src_v.at[j]], rows_v, sem).wait()
            pltpu.sync_copy(rows_v, acc_sh.at[dst_v.at[j]], add=True)
            if with_deg:
                pltpu.sync_copy(ones_v, deg_sh.at[dst_v.at[j]], add=True)
            return carry
        lax.fori_loop(0, CH, step, 0)

        plsc.subcore_barrier()
        pltpu.sync_copy(acc_sh.at[pl.ds(rbase, ROW_SLAB)],
                        acc_out.at[c].at[pl.ds(rbase, ROW_SLAB)])
        if with_deg:
            pltpu.sync_copy(deg_sh.at[pl.ds(rbase, ROW_SLAB)],
                            deg_out.at[c].at[pl.ds(rbase, ROW_SLAB)])

    kern = pl.kernel(body, out_type=tuple(outs), mesh=_MESH,
                     scratch_types=scratch,
                     compiler_params=pltpu.CompilerParams(
                         use_tc_tiling_on_sc=False))
    args = [y, src_p, dst_p, jnp.zeros((BLK, dout), jnp.float32)]
    if with_deg:
        args += [jnp.zeros((BLK, LANES), jnp.float32),
                 jnp.ones((BLK, LANES), jnp.float32)]
    return kern(*args)


_R = 2048


def _tc_first(x, wl, wr, bl):
    din, dout = wl.shape

    def body(x_ref, wl_ref, wr_ref, bl_ref, y_ref, r_ref):
        xb = x_ref[...]
        y_ref[...] = jnp.dot(xb, wl_ref[...], preferred_element_type=jnp.float32)
        r_ref[...] = jnp.dot(xb, wr_ref[...], preferred_element_type=jnp.float32) + bl_ref[...]

    return pl.pallas_call(
        body,
        grid=(N_PAD // _R,),
        in_specs=[
            pl.BlockSpec((_R, din), lambda i: (i, 0)),
            pl.BlockSpec((din, dout), lambda i: (0, 0)),
            pl.BlockSpec((din, dout), lambda i: (0, 0)),
            pl.BlockSpec((1, dout), lambda i: (0, 0)),
        ],
        out_specs=[
            pl.BlockSpec((_R, dout), lambda i: (i, 0)),
            pl.BlockSpec((_R, dout), lambda i: (i, 0)),
        ],
        out_shape=[jax.ShapeDtypeStruct((N_PAD, dout), jnp.float32)] * 2,
    )(x, wl, wr, bl)


def _tc_combine(aggp, degp, r, wl, wr, bl):
    din, dout = wl.shape

    def body(a_ref, d_ref, r_ref, wl_ref, wr_ref, bl_ref, y_ref, r2_ref):
        a = a_ref[0] + a_ref[1]
        deg = d_ref[0] + d_ref[1]
        inv = 1.0 / jnp.maximum(deg[:, 0:1], 1.0)
        h = a * inv + r_ref[...]
        x = 0.5 * h * (1.0 + lax.erf(h * 0.7071067811865476))
        y_ref[...] = jnp.dot(x, wl_ref[...], preferred_element_type=jnp.float32)
        r2_ref[...] = jnp.dot(x, wr_ref[...], preferred_element_type=jnp.float32) + bl_ref[...]

    return pl.pallas_call(
        body,
        grid=(N_PAD // _R,),
        in_specs=[
            pl.BlockSpec((NC, _R, din), lambda i: (0, i, 0)),
            pl.BlockSpec((NC, _R, LANES), lambda i: (0, i, 0)),
            pl.BlockSpec((_R, din), lambda i: (i, 0)),
            pl.BlockSpec((din, dout), lambda i: (0, 0)),
            pl.BlockSpec((din, dout), lambda i: (0, 0)),
            pl.BlockSpec((1, dout), lambda i: (0, 0)),
        ],
        out_specs=[
            pl.BlockSpec((_R, dout), lambda i: (i, 0)),
            pl.BlockSpec((_R, dout), lambda i: (i, 0)),
        ],
        out_shape=[jax.ShapeDtypeStruct((N_PAD, dout), jnp.float32)] * 2,
    )(aggp, degp, r, wl, wr, bl)


def _tc_final(aggp, degp, r):
    dout = r.shape[1]

    def body(a_ref, d_ref, r_ref, o_ref):
        a = a_ref[0] + a_ref[1]
        deg = d_ref[0] + d_ref[1]
        inv = 1.0 / jnp.maximum(deg[:, 0:1], 1.0)
        o_ref[...] = a * inv + r_ref[...]

    return pl.pallas_call(
        body,
        grid=(N_PAD // _R,),
        in_specs=[
            pl.BlockSpec((NC, _R, dout), lambda i: (0, i, 0)),
            pl.BlockSpec((NC, _R, LANES), lambda i: (0, i, 0)),
            pl.BlockSpec((_R, dout), lambda i: (i, 0)),
        ],
        out_specs=pl.BlockSpec((_R, dout), lambda i: (i, 0)),
        out_shape=jax.ShapeDtypeStruct((N_PAD, dout), jnp.float32),
    )(aggp, degp, r)


def kernel(features, edges, Wl0, bl0, Wr0, Wl1, bl1, Wr1,
           Wl2, bl2, Wr2, Wl3, bl3, Wr3):
    f32 = jnp.float32
    src = edges[0].astype(jnp.int32)
    dst = edges[1].astype(jnp.int32)
    src_p = jnp.concatenate(
        [src, jnp.zeros((E_PAD - N_EDGES,), jnp.int32)]).reshape(NW, CH, BLK)
    dst_p = jnp.concatenate(
        [dst, jnp.full((E_PAD - N_EDGES,), N_NODES, jnp.int32)]).reshape(NW, CH, BLK)
    x = jnp.pad(features, ((0, N_PAD - N_NODES), (0, 0)))

    Wl3p = jnp.pad(Wl3, ((0, 0), (0, LANES - 3)))
    Wr3p = jnp.pad(Wr3, ((0, 0), (0, LANES - 3)))
    bl3p = jnp.pad(bl3, ((0, LANES - 3),))

    y, r = _tc_first(x, Wl0, Wr0, bl0.reshape(1, -1))
    aggpa, degp = _agg_call(y[:, :48], src_p, dst_p, True)
    aggpb, = _agg_call(y[:, 48:], src_p, dst_p, False)
    aggp = jnp.concatenate([aggpa, aggpb], axis=2)
    y, r = _tc_combine(aggp, degp, r, Wl1, Wr1, bl1.reshape(1, -1))
    aggp, = _agg_call(y, src_p, dst_p, False)
    y, r = _tc_combine(aggp, degp, r, Wl2, Wr2, bl2.reshape(1, -1))
    aggp, = _agg_call(y, src_p, dst_p, False)
    y, r = _tc_combine(aggp, degp, r, Wl3p, Wr3p, bl3p.reshape(1, -1))
    aggp, = _agg_call(y, src_p, dst_p, False)
    out = _tc_final(aggp, degp, r)
    return out[:N_NODES, :3]


# speedup vs baseline: 1.0907x; 1.0907x over previous
import functools

import jax
import jax.numpy as jnp
from jax import lax
from jax.experimental import pallas as pl
from jax.experimental.pallas import tpu as pltpu
from jax.experimental.pallas import tpu_sc as plsc

N_NODES = 10000
N_EDGES = 320000
N_PAD = 10240
NC, NS, LANES = 2, 16, 16
NW = NC * NS
BLK = 128
CH = 80
E_PAD = NW * CH * BLK
NBUF = 4
ROW_SLAB = N_PAD // NS

_MESH = plsc.VectorSubcoreMesh(core_axis_name="c", subcore_axis_name="s")


def _agg_call(y, src_p, dst_p, with_deg):
    dout = y.shape[1]
    scratch = [
        pltpu.VMEM((CH, BLK), jnp.int32),
        pltpu.VMEM((CH, BLK), jnp.int32),
        pltpu.VMEM((BLK, dout), jnp.float32),
        pltpu.VMEM_SHARED((N_PAD, dout), jnp.float32),
        pltpu.VMEM_SHARED((N_PAD, dout), jnp.float32),
        pltpu.SemaphoreType.DMA,
    ]
    outs = [jax.ShapeDtypeStruct((NC, N_PAD, dout), jnp.float32)]
    if with_deg:
        scratch += [
            pltpu.VMEM((BLK, LANES), jnp.float32),
            pltpu.VMEM_SHARED((N_PAD, LANES), jnp.float32),
        ]
        outs += [jax.ShapeDtypeStruct((NC, N_PAD, LANES), jnp.float32)]

    def body(*refs):
        if with_deg:
            (y_hbm, src_hbm, dst_hbm, zacc_hbm, zdeg_hbm, ones_hbm,
             acc_out, deg_out,
             src_v, dst_v, rows_v, acc_sh, y_sh, sem, ones_v, deg_sh) = refs
        else:
            (y_hbm, src_hbm, dst_hbm, zacc_hbm,
             acc_out,
             src_v, dst_v, rows_v, acc_sh, y_sh, sem) = refs
        c = lax.axis_index("c")
        s = lax.axis_index("s")
        wid = s * NC + c
        rbase = s * ROW_SLAB
        pltpu.sync_copy(src_hbm.at[wid], src_v)
        pltpu.sync_copy(dst_hbm.at[wid], dst_v)
        pltpu.sync_copy(y_hbm.at[pl.ds(rbase, ROW_SLAB)],
                        y_sh.at[pl.ds(rbase, ROW_SLAB)])
        pltpu.sync_copy(zacc_hbm.at[pl.ds(rbase, ROW_SLAB)],
                        acc_sh.at[pl.ds(rbase, ROW_SLAB)])
        if with_deg:
            pltpu.sync_copy(zdeg_hbm.at[pl.ds(rbase, ROW_SLAB)],
                            deg_sh.at[pl.ds(rbase, ROW_SLAB)])
            pltpu.sync_copy(ones_hbm, ones_v)
        plsc.subcore_barrier()

        def step(j, carry):
            pltpu.async_copy(y_sh.at[src_v.at[j]], rows_v, sem).wait()
            pltpu.sync_copy(rows_v, acc_sh.at[dst_v.at[j]], add=True)
            if with_deg:
                pltpu.sync_copy(ones_v, deg_sh.at[dst_v.at[j]], add=True)
            return carry
        lax.fori_loop(0, CH, step, 0)

        plsc.subcore_barrier()
        pltpu.sync_copy(acc_sh.at[pl.ds(rbase, ROW_SLAB)],
                        acc_out.at[c].at[pl.ds(rbase, ROW_SLAB)])
        if with_deg:
            pltpu.sync_copy(deg_sh.at[pl.ds(rbase, ROW_SLAB)],
                            deg_out.at[c].at[pl.ds(rbase, ROW_SLAB)])

    kern = pl.kernel(body, out_type=tuple(outs), mesh=_MESH,
                     scratch_types=scratch,
                     compiler_params=pltpu.CompilerParams(
                         use_tc_tiling_on_sc=False))
    args = [y, src_p, dst_p, jnp.zeros((N_PAD, dout), jnp.float32)]
    if with_deg:
        args += [jnp.zeros((N_PAD, LANES), jnp.float32),
                 jnp.ones((BLK, LANES), jnp.float32)]
    return kern(*args)


_R = 2048


def _tc_first(x, wl, wr, bl):
    din, dout = wl.shape

    def body(x_ref, wl_ref, wr_ref, bl_ref, y_ref, r_ref):
        xb = x_ref[...]
        y_ref[...] = jnp.dot(xb, wl_ref[...], preferred_element_type=jnp.float32)
        r_ref[...] = jnp.dot(xb, wr_ref[...], preferred_element_type=jnp.float32) + bl_ref[...]

    return pl.pallas_call(
        body,
        grid=(N_PAD // _R,),
        in_specs=[
            pl.BlockSpec((_R, din), lambda i: (i, 0)),
            pl.BlockSpec((din, dout), lambda i: (0, 0)),
            pl.BlockSpec((din, dout), lambda i: (0, 0)),
            pl.BlockSpec((1, dout), lambda i: (0, 0)),
        ],
        out_specs=[
            pl.BlockSpec((_R, dout), lambda i: (i, 0)),
            pl.BlockSpec((_R, dout), lambda i: (i, 0)),
        ],
        out_shape=[jax.ShapeDtypeStruct((N_PAD, dout), jnp.float32)] * 2,
    )(x, wl, wr, bl)


def _tc_combine(aggps, degp, r, wl, wr, bl):
    din, dout = wl.shape
    widths = [a.shape[2] for a in aggps]

    def body(*refs):
        a_refs = refs[:len(widths)]
        d_ref, r_ref, wl_ref, wr_ref, bl_ref, y_ref, r2_ref = refs[len(widths):]
        a = jnp.concatenate([ar[0] + ar[1] for ar in a_refs], axis=-1)
        deg = d_ref[0] + d_ref[1]
        inv = 1.0 / jnp.maximum(deg[:, 0:1], 1.0)
        h = a * inv + r_ref[...]
        x = 0.5 * h * (1.0 + lax.erf(h * 0.7071067811865476))
        y_ref[...] = jnp.dot(x, wl_ref[...], preferred_element_type=jnp.float32)
        r2_ref[...] = jnp.dot(x, wr_ref[...], preferred_element_type=jnp.float32) + bl_ref[...]

    return pl.pallas_call(
        body,
        grid=(N_PAD // _R,),
        in_specs=[pl.BlockSpec((NC, _R, w), lambda i: (0, i, 0))
                  for w in widths] + [
            pl.BlockSpec((NC, _R, LANES), lambda i: (0, i, 0)),
            pl.BlockSpec((_R, din), lambda i: (i, 0)),
            pl.BlockSpec((din, dout), lambda i: (0, 0)),
            pl.BlockSpec((din, dout), lambda i: (0, 0)),
            pl.BlockSpec((1, dout), lambda i: (0, 0)),
        ],
        out_specs=[
            pl.BlockSpec((_R, dout), lambda i: (i, 0)),
            pl.BlockSpec((_R, dout), lambda i: (i, 0)),
        ],
        out_shape=[jax.ShapeDtypeStruct((N_PAD, dout), jnp.float32)] * 2,
    )(*aggps, degp, r, wl, wr, bl)


def _tc_final(aggp, degp, r):
    dout = r.shape[1]

    def body(a_ref, d_ref, r_ref, o_ref):
        a = a_ref[0] + a_ref[1]
        deg = d_ref[0] + d_ref[1]
        inv = 1.0 / jnp.maximum(deg[:, 0:1], 1.0)
        o_ref[...] = a * inv + r_ref[...]

    return pl.pallas_call(
        body,
        grid=(N_PAD // _R,),
        in_specs=[
            pl.BlockSpec((NC, _R, dout), lambda i: (0, i, 0)),
            pl.BlockSpec((NC, _R, LANES), lambda i: (0, i, 0)),
            pl.BlockSpec((_R, dout), lambda i: (i, 0)),
        ],
        out_specs=pl.BlockSpec((_R, dout), lambda i: (i, 0)),
        out_shape=jax.ShapeDtypeStruct((N_PAD, dout), jnp.float32),
    )(aggp, degp, r)


def kernel(features, edges, Wl0, bl0, Wr0, Wl1, bl1, Wr1,
           Wl2, bl2, Wr2, Wl3, bl3, Wr3):
    f32 = jnp.float32
    src = edges[0].astype(jnp.int32)
    dst = edges[1].astype(jnp.int32)
    src_p = jnp.concatenate(
        [src, jnp.zeros((E_PAD - N_EDGES,), jnp.int32)]).reshape(NW, CH, BLK)
    dst_p = jnp.concatenate(
        [dst, jnp.full((E_PAD - N_EDGES,), N_NODES, jnp.int32)]).reshape(NW, CH, BLK)
    x = jnp.pad(features, ((0, N_PAD - N_NODES), (0, 0)))

    Wl3p = jnp.pad(Wl3, ((0, 0), (0, LANES - 3)))
    Wr3p = jnp.pad(Wr3, ((0, 0), (0, LANES - 3)))
    bl3p = jnp.pad(bl3, ((0, LANES - 3),))

    y, r = _tc_first(x, Wl0, Wr0, bl0.reshape(1, -1))
    aggpa, degp = _agg_call(y[:, :48], src_p, dst_p, True)
    aggpb, = _agg_call(y[:, 48:], src_p, dst_p, False)
    y, r = _tc_combine([aggpa, aggpb], degp, r, Wl1, Wr1, bl1.reshape(1, -1))
    aggp, = _agg_call(y, src_p, dst_p, False)
    y, r = _tc_combine([aggp], degp, r, Wl2, Wr2, bl2.reshape(1, -1))
    aggp, = _agg_call(y, src_p, dst_p, False)
    y, r = _tc_combine([aggp], degp, r, Wl3p, Wr3p, bl3p.reshape(1, -1))
    aggp, = _agg_call(y, src_p, dst_p, False)
    out = _tc_final(aggp, degp, r)
    return out[:N_NODES, :3]
